# baseline (speedup 1.0000x reference)
import math

import jax
import jax.numpy as jnp
from jax.experimental import pallas as pl
from jax.experimental.pallas import tpu as pltpu
from jax.experimental.pallas import tpu_sc as plsc

_NF = 16
_LAT = 3.0
_K = 16
_BLK = 256
_SC_LANES = 16


def _topk_body(qx_ref, qy_ref, px_ref, py_ref, idx_ref, gidx_ref):
    b = pl.program_id(0)
    j = pl.program_id(1)
    blk = qx_ref.shape[1]
    L = px_ref.shape[2]
    qx = qx_ref[0]
    qy = qy_ref[0]
    px = px_ref[0]
    py = py_ref[0]
    dx = qx - px
    dy = qy - py
    d2 = dx * dx + dy * dy
    col = jax.lax.broadcasted_iota(jnp.int32, (blk, L), 1)
    rowg = j * blk + jax.lax.broadcasted_iota(jnp.int32, (blk, L), 0)
    inf = jnp.float32(jnp.inf)
    work = jnp.where(col == rowg, inf, d2)
    idxs = []
    m_prev = jnp.full((blk, 1), -jnp.inf, jnp.float32)
    am_prev = jnp.full((blk, 1), -1, jnp.int32)
    for _ in range(_K):
        elig = (work > m_prev) | ((work == m_prev) & (col > am_prev))
        m = jnp.min(jnp.where(elig, work, inf), axis=1, keepdims=True)
        am = jnp.min(jnp.where(elig & (work == m), col, L),
                     axis=1, keepdims=True)
        idxs.append(am)
        m_prev, am_prev = m, am
    idx = jnp.concatenate(idxs, axis=1)
    idx_ref[0] = idx
    gidx_ref[0] = idx + b * L


def _gather_body(gidx_hbm, pxf_hbm, pyf_hbm, nx_hbm, ny_hbm,
                 idx_v, px_v, py_v, nx_v, ny_v):
    wid = jax.lax.axis_index("s") * 2 + jax.lax.axis_index("c")
    n_per = idx_v.shape[0]
    base = wid * n_per
    pltpu.sync_copy(gidx_hbm.at[pl.ds(base, n_per)], idx_v)
    pltpu.sync_copy(pxf_hbm, px_v)
    pltpu.sync_copy(pyf_hbm, py_v)

    def body(i, carry):
        off = pl.multiple_of(i * _SC_LANES, _SC_LANES)
        iv = idx_v[pl.ds(off, _SC_LANES)]
        nx_v[pl.ds(off, _SC_LANES)] = plsc.load_gather(px_v, [iv])
        ny_v[pl.ds(off, _SC_LANES)] = plsc.load_gather(py_v, [iv])
        return carry

    jax.lax.fori_loop(0, n_per // _SC_LANES, body, 0)
    pltpu.sync_copy(nx_v, nx_hbm.at[pl.ds(base, n_per)])
    pltpu.sync_copy(ny_v, ny_hbm.at[pl.ds(base, n_per)])


def _rpe_body(qx_ref, qy_ref, nx_ref, ny_ref, fr_ref, rpe_ref, dst_ref):
    qx = qx_ref[0]
    qy = qy_ref[0]
    nx = nx_ref[0]
    ny = ny_ref[0]
    dxk = nx - qx
    dyk = ny - qy
    dst_ref[0] = jnp.sqrt(dxk * dxk + dyk * dyk + 1e-8)
    fr = fr_ref[...].reshape(1, 1, _NF)
    fx = dxk[:, :, None] * fr
    fy = dyk[:, :, None] * fr
    half_pi = jnp.float32(math.pi / 2)
    phases = jnp.concatenate([fx, fx + half_pi, fy, fy + half_pi], axis=-1)
    rpe_ref[0] = jnp.sin(phases)


def kernel(positions, k):
    B, L, _ = positions.shape
    k_static = min(_K, L - 1)
    physical_scale = _LAT * math.sqrt(k_static / math.pi)
    freqs = (2.0 ** jnp.arange(_NF, dtype=jnp.float32)) * (math.pi / physical_scale)
    freqs2 = freqs.reshape(1, _NF)
    px = positions[:, :, 0]
    py = positions[:, :, 1]
    pxq = px.reshape(B, L, 1)
    pyq = py.reshape(B, L, 1)
    pxr = px.reshape(B, 1, L)
    pyr = py.reshape(B, 1, L)
    grid = (B, L // _BLK)

    idx, gidx = pl.pallas_call(
        _topk_body, grid=grid,
        in_specs=[
            pl.BlockSpec((1, _BLK, 1), lambda b, j: (b, j, 0)),
            pl.BlockSpec((1, _BLK, 1), lambda b, j: (b, j, 0)),
            pl.BlockSpec((1, 1, L), lambda b, j: (b, 0, 0)),
            pl.BlockSpec((1, 1, L), lambda b, j: (b, 0, 0)),
        ],
        out_specs=(
            pl.BlockSpec((1, _BLK, _K), lambda b, j: (b, j, 0)),
            pl.BlockSpec((1, _BLK, _K), lambda b, j: (b, j, 0)),
        ),
        out_shape=(
            jax.ShapeDtypeStruct((B, L, _K), jnp.int32),
            jax.ShapeDtypeStruct((B, L, _K), jnp.int32),
        ))(pxq, pyq, pxr, pyr)

    N = B * L * _K
    n_per = N // 32
    mesh = plsc.VectorSubcoreMesh(core_axis_name="c", subcore_axis_name="s")
    nxf, nyf = pl.kernel(
        _gather_body,
        out_type=(
            jax.ShapeDtypeStruct((N,), jnp.float32),
            jax.ShapeDtypeStruct((N,), jnp.float32),
        ),
        mesh=mesh,
        compiler_params=pltpu.CompilerParams(needs_layout_passes=False),
        scratch_types=[
            pltpu.VMEM((n_per,), jnp.int32),
            pltpu.VMEM((B * L,), jnp.float32),
            pltpu.VMEM((B * L,), jnp.float32),
            pltpu.VMEM((n_per,), jnp.float32),
            pltpu.VMEM((n_per,), jnp.float32),
        ],
    )(gidx.reshape(N), px.reshape(B * L), py.reshape(B * L))
    nxk = nxf.reshape(B, L, _K)
    nyk = nyf.reshape(B, L, _K)

    rpe, dst = pl.pallas_call(
        _rpe_body, grid=grid,
        in_specs=[
            pl.BlockSpec((1, _BLK, 1), lambda b, j: (b, j, 0)),
            pl.BlockSpec((1, _BLK, 1), lambda b, j: (b, j, 0)),
            pl.BlockSpec((1, _BLK, _K), lambda b, j: (b, j, 0)),
            pl.BlockSpec((1, _BLK, _K), lambda b, j: (b, j, 0)),
            pl.BlockSpec((1, _NF), lambda b, j: (0, 0)),
        ],
        out_specs=(
            pl.BlockSpec((1, _BLK, _K, 4 * _NF), lambda b, j: (b, j, 0, 0)),
            pl.BlockSpec((1, _BLK, _K), lambda b, j: (b, j, 0)),
        ),
        out_shape=(
            jax.ShapeDtypeStruct((B, L, _K, 4 * _NF), jnp.float32),
            jax.ShapeDtypeStruct((B, L, _K), jnp.float32),
        ))(pxq, pyq, nxk, nyk, freqs2)

    neighbor_positions = jnp.stack([nxk, nyk], axis=-1)
    pat = jnp.concatenate([
        jnp.zeros((_NF,), jnp.float32), jnp.ones((_NF,), jnp.float32),
        jnp.zeros((_NF,), jnp.float32), jnp.ones((_NF,), jnp.float32)])
    self_rpe = jnp.broadcast_to(pat, (B, L, 1, 4 * _NF))
    return (idx, rpe, self_rpe, dst, neighbor_positions)


# speedup vs baseline: 1.6750x; 1.6750x over previous
import math

import jax
import jax.numpy as jnp
from jax.experimental import pallas as pl
from jax.experimental.pallas import tpu as pltpu
from jax.experimental.pallas import tpu_sc as plsc

_NF = 16
_LAT = 3.0
_K = 16
_BLK = 256
_SC_LANES = 16


def _topk_body(qx_ref, qy_ref, px_ref, py_ref, idx_ref, gidx_ref):
    b = pl.program_id(0)
    j = pl.program_id(1)
    blk = qx_ref.shape[1]
    L = px_ref.shape[2]
    qx = qx_ref[0]
    qy = qy_ref[0]
    px = px_ref[0]
    py = py_ref[0]
    dx = qx - px
    dy = qy - py
    d2 = dx * dx + dy * dy
    col = jax.lax.broadcasted_iota(jnp.int32, (blk, L), 1)
    rowg = j * blk + jax.lax.broadcasted_iota(jnp.int32, (blk, L), 0)
    inf = jnp.float32(jnp.inf)
    work = jnp.where(col == rowg, inf, d2)
    idxs = []
    for _ in range(_K):
        m = jnp.min(work, axis=1, keepdims=True)
        am = jnp.min(jnp.where(work == m, col, L), axis=1, keepdims=True)
        work = jnp.where(col == am, inf, work)
        idxs.append(am)
    idx = jnp.concatenate(idxs, axis=1)
    idx_ref[0] = idx
    gidx_ref[0] = idx + b * L


def _gather_body(gidx_hbm, pxf_hbm, pyf_hbm, nx_hbm, ny_hbm,
                 idx_v, px_v, py_v, nx_v, ny_v):
    wid = jax.lax.axis_index("s") * 2 + jax.lax.axis_index("c")
    n_per = idx_v.shape[0]
    base = wid * n_per
    pltpu.sync_copy(gidx_hbm.at[pl.ds(base, n_per)], idx_v)
    pltpu.sync_copy(pxf_hbm, px_v)
    pltpu.sync_copy(pyf_hbm, py_v)

    def body(i, carry):
        off = pl.multiple_of(i * _SC_LANES, _SC_LANES)
        iv = idx_v[pl.ds(off, _SC_LANES)]
        nx_v[pl.ds(off, _SC_LANES)] = plsc.load_gather(px_v, [iv])
        ny_v[pl.ds(off, _SC_LANES)] = plsc.load_gather(py_v, [iv])
        return carry

    jax.lax.fori_loop(0, n_per // _SC_LANES, body, 0)
    pltpu.sync_copy(nx_v, nx_hbm.at[pl.ds(base, n_per)])
    pltpu.sync_copy(ny_v, ny_hbm.at[pl.ds(base, n_per)])


def _rpe_body(qx_ref, qy_ref, nx_ref, ny_ref, sel_ref, fo_ref, rpe_ref, dst_ref):
    qx = qx_ref[0]
    qy = qy_ref[0]
    nx = nx_ref[0]
    ny = ny_ref[0]
    dxk = nx - qx
    dyk = ny - qy
    dst_ref[0] = jnp.sqrt(dxk * dxk + dyk * dyk + 1e-8)
    dxy = jnp.concatenate([dxk, dyk], axis=1)
    d_flat = jax.lax.dot_general(
        dxy, sel_ref[...],
        (((1,), (0,)), ((), ())),
        precision=jax.lax.Precision.HIGHEST,
        preferred_element_type=jnp.float32)
    fo = fo_ref[...]
    phases = d_flat * fo[0:1, :] + fo[1:2, :]
    rpe_ref[0] = jnp.sin(phases)


def kernel(positions, k):
    B, L, _ = positions.shape
    k_static = min(_K, L - 1)
    physical_scale = _LAT * math.sqrt(k_static / math.pi)
    freqs = (2.0 ** jnp.arange(_NF, dtype=jnp.float32)) * (math.pi / physical_scale)
    freqs2 = freqs.reshape(1, _NF)
    px = positions[:, :, 0]
    py = positions[:, :, 1]
    pxq = px.reshape(B, L, 1)
    pyq = py.reshape(B, L, 1)
    pxr = px.reshape(B, 1, L)
    pyr = py.reshape(B, 1, L)
    grid = (B, L // _BLK)

    idx, gidx = pl.pallas_call(
        _topk_body, grid=grid,
        in_specs=[
            pl.BlockSpec((1, _BLK, 1), lambda b, j: (b, j, 0)),
            pl.BlockSpec((1, _BLK, 1), lambda b, j: (b, j, 0)),
            pl.BlockSpec((1, 1, L), lambda b, j: (b, 0, 0)),
            pl.BlockSpec((1, 1, L), lambda b, j: (b, 0, 0)),
        ],
        out_specs=(
            pl.BlockSpec((1, _BLK, _K), lambda b, j: (b, j, 0)),
            pl.BlockSpec((1, _BLK, _K), lambda b, j: (b, j, 0)),
        ),
        out_shape=(
            jax.ShapeDtypeStruct((B, L, _K), jnp.int32),
            jax.ShapeDtypeStruct((B, L, _K), jnp.int32),
        ))(pxq, pyq, pxr, pyr)

    N = B * L * _K
    n_per = N // 32
    mesh = plsc.VectorSubcoreMesh(core_axis_name="c", subcore_axis_name="s")
    nxf, nyf = pl.kernel(
        _gather_body,
        out_type=(
            jax.ShapeDtypeStruct((N,), jnp.float32),
            jax.ShapeDtypeStruct((N,), jnp.float32),
        ),
        mesh=mesh,
        compiler_params=pltpu.CompilerParams(needs_layout_passes=False),
        scratch_types=[
            pltpu.VMEM((n_per,), jnp.int32),
            pltpu.VMEM((B * L,), jnp.float32),
            pltpu.VMEM((B * L,), jnp.float32),
            pltpu.VMEM((n_per,), jnp.float32),
            pltpu.VMEM((n_per,), jnp.float32),
        ],
    )(gidx.reshape(N), px.reshape(B * L), py.reshape(B * L))
    nxk = nxf.reshape(B, L, _K)
    nyk = nyf.reshape(B, L, _K)

    F = 4 * _NF * _K
    lane = jnp.arange(F, dtype=jnp.int32)
    kk = lane // (4 * _NF)
    cl = lane % (4 * _NF)
    grp = cl // _NF
    aa = cl % _NF
    src_col = jnp.where(grp < 2, kk, _K + kk)
    sel = (src_col[None, :] == jnp.arange(2 * _K, dtype=jnp.int32)[:, None]
           ).astype(jnp.float32)
    fmul = freqs[aa]
    foff = jnp.where((grp % 2) == 1, jnp.float32(math.pi / 2), 0.0)
    fo = jnp.stack([fmul, foff], axis=0)

    rpe_flat, dst = pl.pallas_call(
        _rpe_body, grid=grid,
        in_specs=[
            pl.BlockSpec((1, _BLK, 1), lambda b, j: (b, j, 0)),
            pl.BlockSpec((1, _BLK, 1), lambda b, j: (b, j, 0)),
            pl.BlockSpec((1, _BLK, _K), lambda b, j: (b, j, 0)),
            pl.BlockSpec((1, _BLK, _K), lambda b, j: (b, j, 0)),
            pl.BlockSpec((2 * _K, F), lambda b, j: (0, 0)),
            pl.BlockSpec((2, F), lambda b, j: (0, 0)),
        ],
        out_specs=(
            pl.BlockSpec((1, _BLK, F), lambda b, j: (b, j, 0)),
            pl.BlockSpec((1, _BLK, _K), lambda b, j: (b, j, 0)),
        ),
        out_shape=(
            jax.ShapeDtypeStruct((B, L, F), jnp.float32),
            jax.ShapeDtypeStruct((B, L, _K), jnp.float32),
        ))(pxq, pyq, nxk, nyk, sel, fo)
    rpe = rpe_flat.reshape(B, L, _K, 4 * _NF)

    neighbor_positions = jnp.stack([nxk, nyk], axis=-1)
    pat = jnp.concatenate([
        jnp.zeros((_NF,), jnp.float32), jnp.ones((_NF,), jnp.float32),
        jnp.zeros((_NF,), jnp.float32), jnp.ones((_NF,), jnp.float32)])
    self_rpe = jnp.broadcast_to(pat, (B, L, 1, 4 * _NF))
    return (idx, rpe, self_rpe, dst, neighbor_positions)


# speedup vs baseline: 1.7811x; 1.0633x over previous
import math

import jax
import jax.numpy as jnp
from jax.experimental import pallas as pl
from jax.experimental.pallas import tpu as pltpu
from jax.experimental.pallas import tpu_sc as plsc

_NF = 16
_LAT = 3.0
_K = 16
_BLK = 512
_SC_LANES = 16


def _topk_body(qx_ref, qy_ref, px_ref, py_ref, idx_ref, gidx_ref):
    b = pl.program_id(0)
    j = pl.program_id(1)
    blk = qx_ref.shape[1]
    L = px_ref.shape[2]
    qx = qx_ref[0]
    qy = qy_ref[0]
    px = px_ref[0]
    py = py_ref[0]
    dx = qx - px
    dy = qy - py
    d2 = dx * dx + dy * dy
    col = jax.lax.broadcasted_iota(jnp.int32, (blk, L), 1)
    rowg = j * blk + jax.lax.broadcasted_iota(jnp.int32, (blk, L), 0)
    inf = jnp.float32(jnp.inf)
    work = jnp.where(col == rowg, inf, d2)
    idxs = []
    for _ in range(_K):
        m = jnp.min(work, axis=1, keepdims=True)
        am = jnp.min(jnp.where(work == m, col, L), axis=1, keepdims=True)
        work = jnp.where(col == am, inf, work)
        idxs.append(am)
    idx = jnp.concatenate(idxs, axis=1)
    idx_ref[0] = idx
    gidx_ref[0] = idx + b * L


def _gather_body(gidx_hbm, pxf_hbm, pyf_hbm, nx_hbm, ny_hbm,
                 idx_v, px_v, py_v, nx_v, ny_v):
    wid = jax.lax.axis_index("s") * 2 + jax.lax.axis_index("c")
    n_per = idx_v.shape[0]
    base = wid * n_per
    pltpu.sync_copy(gidx_hbm.at[pl.ds(base, n_per)], idx_v)
    pltpu.sync_copy(pxf_hbm, px_v)
    pltpu.sync_copy(pyf_hbm, py_v)

    def body(i, carry):
        off = pl.multiple_of(i * _SC_LANES, _SC_LANES)
        iv = idx_v[pl.ds(off, _SC_LANES)]
        nx_v[pl.ds(off, _SC_LANES)] = plsc.load_gather(px_v, [iv])
        ny_v[pl.ds(off, _SC_LANES)] = plsc.load_gather(py_v, [iv])
        return carry

    jax.lax.fori_loop(0, n_per // _SC_LANES, body, 0)
    pltpu.sync_copy(nx_v, nx_hbm.at[pl.ds(base, n_per)])
    pltpu.sync_copy(ny_v, ny_hbm.at[pl.ds(base, n_per)])


def _rpe_body(qx_ref, qy_ref, nx_ref, ny_ref, sel_ref, fo_ref, rpe_ref, dst_ref):
    qx = qx_ref[0]
    qy = qy_ref[0]
    nx = nx_ref[0]
    ny = ny_ref[0]
    dxk = nx - qx
    dyk = ny - qy
    dst_ref[0] = jnp.sqrt(dxk * dxk + dyk * dyk + 1e-8)
    dxy = jnp.concatenate([dxk, dyk], axis=1)
    d_flat = jax.lax.dot_general(
        dxy, sel_ref[...],
        (((1,), (0,)), ((), ())),
        precision=jax.lax.Precision.HIGHEST,
        preferred_element_type=jnp.float32)
    fo = fo_ref[...]
    phases = d_flat * fo[0:1, :] + fo[1:2, :]
    rpe_ref[0] = jnp.sin(phases)


def kernel(positions, k):
    B, L, _ = positions.shape
    k_static = min(_K, L - 1)
    physical_scale = _LAT * math.sqrt(k_static / math.pi)
    freqs = (2.0 ** jnp.arange(_NF, dtype=jnp.float32)) * (math.pi / physical_scale)
    freqs2 = freqs.reshape(1, _NF)
    px = positions[:, :, 0]
    py = positions[:, :, 1]
    pxq = px.reshape(B, L, 1)
    pyq = py.reshape(B, L, 1)
    pxr = px.reshape(B, 1, L)
    pyr = py.reshape(B, 1, L)
    grid = (B, L // _BLK)

    idx, gidx = pl.pallas_call(
        _topk_body, grid=grid,
        in_specs=[
            pl.BlockSpec((1, _BLK, 1), lambda b, j: (b, j, 0)),
            pl.BlockSpec((1, _BLK, 1), lambda b, j: (b, j, 0)),
            pl.BlockSpec((1, 1, L), lambda b, j: (b, 0, 0)),
            pl.BlockSpec((1, 1, L), lambda b, j: (b, 0, 0)),
        ],
        out_specs=(
            pl.BlockSpec((1, _BLK, _K), lambda b, j: (b, j, 0)),
            pl.BlockSpec((1, _BLK, _K), lambda b, j: (b, j, 0)),
        ),
        out_shape=(
            jax.ShapeDtypeStruct((B, L, _K), jnp.int32),
            jax.ShapeDtypeStruct((B, L, _K), jnp.int32),
        ))(pxq, pyq, pxr, pyr)

    N = B * L * _K
    n_per = N // 32
    mesh = plsc.VectorSubcoreMesh(core_axis_name="c", subcore_axis_name="s")
    nxf, nyf = pl.kernel(
        _gather_body,
        out_type=(
            jax.ShapeDtypeStruct((N,), jnp.float32),
            jax.ShapeDtypeStruct((N,), jnp.float32),
        ),
        mesh=mesh,
        compiler_params=pltpu.CompilerParams(needs_layout_passes=False),
        scratch_types=[
            pltpu.VMEM((n_per,), jnp.int32),
            pltpu.VMEM((B * L,), jnp.float32),
            pltpu.VMEM((B * L,), jnp.float32),
            pltpu.VMEM((n_per,), jnp.float32),
            pltpu.VMEM((n_per,), jnp.float32),
        ],
    )(gidx.reshape(N), px.reshape(B * L), py.reshape(B * L))
    nxk = nxf.reshape(B, L, _K)
    nyk = nyf.reshape(B, L, _K)

    F = 4 * _NF * _K
    lane = jnp.arange(F, dtype=jnp.int32)
    kk = lane // (4 * _NF)
    cl = lane % (4 * _NF)
    grp = cl // _NF
    aa = cl % _NF
    src_col = jnp.where(grp < 2, kk, _K + kk)
    sel = (src_col[None, :] == jnp.arange(2 * _K, dtype=jnp.int32)[:, None]
           ).astype(jnp.float32)
    fmul = freqs[aa]
    foff = jnp.where((grp % 2) == 1, jnp.float32(math.pi / 2), 0.0)
    fo = jnp.stack([fmul, foff], axis=0)

    rpe_flat, dst = pl.pallas_call(
        _rpe_body, grid=grid,
        in_specs=[
            pl.BlockSpec((1, _BLK, 1), lambda b, j: (b, j, 0)),
            pl.BlockSpec((1, _BLK, 1), lambda b, j: (b, j, 0)),
            pl.BlockSpec((1, _BLK, _K), lambda b, j: (b, j, 0)),
            pl.BlockSpec((1, _BLK, _K), lambda b, j: (b, j, 0)),
            pl.BlockSpec((2 * _K, F), lambda b, j: (0, 0)),
            pl.BlockSpec((2, F), lambda b, j: (0, 0)),
        ],
        out_specs=(
            pl.BlockSpec((1, _BLK, F), lambda b, j: (b, j, 0)),
            pl.BlockSpec((1, _BLK, _K), lambda b, j: (b, j, 0)),
        ),
        out_shape=(
            jax.ShapeDtypeStruct((B, L, F), jnp.float32),
            jax.ShapeDtypeStruct((B, L, _K), jnp.float32),
        ))(pxq, pyq, nxk, nyk, sel, fo)
    rpe = rpe_flat.reshape(B, L, _K, 4 * _NF)

    neighbor_positions = jnp.stack([nxk, nyk], axis=-1)
    pat = jnp.concatenate([
        jnp.zeros((_NF,), jnp.float32), jnp.ones((_NF,), jnp.float32),
        jnp.zeros((_NF,), jnp.float32), jnp.ones((_NF,), jnp.float32)])
    self_rpe = jnp.broadcast_to(pat, (B, L, 1, 4 * _NF))
    return (idx, rpe, self_rpe, dst, neighbor_positions)
